# baseline (speedup 1.0000x reference)
import functools

import jax
import jax.numpy as jnp
from jax import lax
from jax.experimental import pallas as pl
from jax.experimental.pallas import tpu as pltpu
from jax.experimental.pallas import tpu_sc as plsc

VOCAB = 1000
D_POS = 128
D_OUT = VOCAB + D_POS
LANES = 16


def kernel(X, position_embeddings):
    batch, length = X.shape
    pos_pad = jnp.pad(position_embeddings, ((0, 0), (VOCAB, 0)))
    lpad = (length + LANES - 1) // LANES * LANES
    x_pad = jnp.pad(X, ((0, 0), (0, lpad - length)))

    info = plsc.get_sparse_core_info()
    nw = info.num_cores * info.num_subcores
    b_per_w = batch // nw
    nchunk = lpad // LANES

    mesh = plsc.VectorSubcoreMesh(core_axis_name="c", subcore_axis_name="s")

    @functools.partial(
        pl.kernel,
        out_type=jax.ShapeDtypeStruct((batch, length, D_OUT), jnp.float32),
        mesh=mesh,
        compiler_params=pltpu.CompilerParams(
            use_tc_tiling_on_sc=False, needs_layout_passes=False
        ),
        scratch_types=[
            pltpu.VMEM((b_per_w, lpad), jnp.int32),
            pltpu.VMEM((2, length, D_OUT), jnp.float32),
            pltpu.SemaphoreType.DMA((2,)),
        ],
    )
    def run(x_hbm, pos_hbm, out_hbm, xv, buf, sems):
        wid = lax.axis_index("s") * info.num_cores + lax.axis_index("c")
        base = wid * b_per_w
        pltpu.sync_copy(x_hbm.at[pl.ds(base, b_per_w)], xv)
        pltpu.sync_copy(pos_hbm, buf.at[0])
        pltpu.sync_copy(pos_hbm, buf.at[1])

        ones = jnp.full((LANES,), 1.0, jnp.float32)
        zeros = jnp.zeros((LANES,), jnp.float32)
        lane = lax.broadcasted_iota(jnp.int32, (LANES,), 0)

        def scatter(s, r, val):
            for k in range(nchunk):
                rows = lane + (k * LANES)
                toks = xv[r, pl.ds(k * LANES, LANES)]
                if (k + 1) * LANES <= length:
                    plsc.store_scatter(buf.at[s], [rows, toks], val)
                else:
                    mask = rows < length
                    plsc.store_scatter(buf.at[s], [rows, toks], val, mask=mask)

        copies = [None, None]
        for r in range(b_per_w):
            s = r % 2
            if copies[s] is not None:
                copies[s].wait()
                scatter(s, r - 2, zeros)
            scatter(s, r, ones)
            copies[s] = pltpu.async_copy(buf.at[s], out_hbm.at[base + r], sems.at[s])
        copies[(b_per_w - 1) % 2].wait()
        copies[b_per_w % 2].wait()

    return run(x_pad, pos_pad)


# speedup vs baseline: 2.0111x; 2.0111x over previous
import functools

import jax
import jax.numpy as jnp
from jax import lax
from jax.experimental import pallas as pl
from jax.experimental.pallas import tpu as pltpu
from jax.experimental.pallas import tpu_sc as plsc

VOCAB = 1000
D_POS = 128
D_OUT = VOCAB + D_POS
LANES = 16


def kernel(X, position_embeddings):
    batch, length = X.shape
    pos_pad = jnp.pad(position_embeddings, ((0, 0), (VOCAB, 0)))
    lpad = (length + LANES - 1) // LANES * LANES
    x_pad = jnp.pad(X, ((0, 0), (0, lpad - length)))

    info = plsc.get_sparse_core_info()
    nw = info.num_cores * info.num_subcores
    b_per_w = batch // nw
    nchunk = lpad // LANES

    mesh = plsc.VectorSubcoreMesh(core_axis_name="c", subcore_axis_name="s")

    @functools.partial(
        pl.kernel,
        out_type=jax.ShapeDtypeStruct((batch, length, D_OUT), jnp.float32),
        mesh=mesh,
        compiler_params=pltpu.CompilerParams(
            use_tc_tiling_on_sc=True, needs_layout_passes=False
        ),
        scratch_types=[
            pltpu.VMEM((b_per_w, lpad), jnp.int32),
            pltpu.VMEM((length, D_OUT), jnp.float32),
            pltpu.SemaphoreType.DMA,
        ],
    )
    def run(x_hbm, pos_hbm, out_hbm, xv, buf, sem):
        wid = lax.axis_index("s") * info.num_cores + lax.axis_index("c")
        base = wid * b_per_w
        pltpu.sync_copy(x_hbm.at[pl.ds(base, b_per_w)], xv)
        pltpu.sync_copy(pos_hbm, buf)

        ones = jnp.full((LANES,), 1.0, jnp.float32)
        zeros = jnp.zeros((LANES,), jnp.float32)
        lane = lax.broadcasted_iota(jnp.int32, (LANES,), 0)

        def scatter(r, val):
            for k in range(nchunk):
                rows = lane + (k * LANES)
                toks = xv[r, pl.ds(k * LANES, LANES)]
                if (k + 1) * LANES <= length:
                    plsc.store_scatter(buf, [rows, toks], val)
                else:
                    mask = rows < length
                    plsc.store_scatter(buf, [rows, toks], val, mask=mask)

        for r in range(b_per_w):
            scatter(r, ones)
            pltpu.async_copy(buf, out_hbm.at[base + r], sem).wait()
            scatter(r, zeros)

    return run(x_pad, pos_pad)
